# baseline (speedup 1.0000x reference)
import functools

import jax
import jax.numpy as jnp
from jax import lax
from jax.experimental import pallas as pl
from jax.experimental.pallas import tpu as pltpu
from jax.experimental.pallas import tpu_sc as plsc

DIM = 32
EPS = 1e-5

NC = 2
NS = 16
LANES = 16
NW = NC * NS

SUB = 512
SUBS_PER_BLK = 1
R = SUB * SUBS_PER_BLK
NBUF = 4

TC_BLK = 16384


def _gather_body(x_hbm, table_hbm, out_hbm, idx_v, rows_v, sem_g, sem_w,
                 sem_i, n_rows_per_worker):
    wid = lax.axis_index("s") * NC + lax.axis_index("c")
    nb = n_rows_per_worker // R
    sub_base0 = wid * (n_rows_per_worker // SUB)

    def idx_src(b):
        return x_hbm.at[pl.ds(sub_base0 + b * SUBS_PER_BLK, SUBS_PER_BLK)]

    def fire_gathers(b):
        q = b & (NBUF - 1)
        for j in range(SUBS_PER_BLK):
            pltpu.async_copy(table_hbm.at[idx_v.at[q, j]],
                             rows_v.at[q, j], sem_g)

    def wait_gathers(b):
        p = b & (NBUF - 1)
        for j in range(SUBS_PER_BLK):
            pltpu.make_async_copy(table_hbm.at[idx_v.at[p, j]],
                                  rows_v.at[p, j], sem_g).wait()

    def wait_writeout():
        pltpu.make_async_copy(
            rows_v.at[0], out_hbm.at[pl.ds(sub_base0, SUBS_PER_BLK)],
            sem_w).wait()

    pltpu.sync_copy(idx_src(0), idx_v.at[0])
    pltpu.async_copy(idx_src(1), idx_v.at[1], sem_i)
    fire_gathers(0)

    def block(b, carry):
        p = b & (NBUF - 1)

        @pl.when(b + 1 < nb)
        def _prefetch():
            @pl.when(b >= NBUF - 1)
            def _():
                wait_writeout()
            pltpu.make_async_copy(idx_src(b + 1),
                                  idx_v.at[(b + 1) & (NBUF - 1)],
                                  sem_i).wait()

            @pl.when(b + 2 < nb)
            def _():
                pltpu.async_copy(idx_src(b + 2),
                                 idx_v.at[(b + 2) & (NBUF - 1)], sem_i)
            fire_gathers(b + 1)

        wait_gathers(b)
        pltpu.async_copy(
            rows_v.at[p],
            out_hbm.at[pl.ds(sub_base0 + b * SUBS_PER_BLK, SUBS_PER_BLK)],
            sem_w)
        return carry

    lax.fori_loop(0, nb, block, None, unroll=False)
    for _ in range(NBUF):
        wait_writeout()


def _sc_gather(x2, table, n):
    n_per_worker = n // NW
    dt = table.dtype
    mesh = plsc.VectorSubcoreMesh(core_axis_name="c", subcore_axis_name="s",
                                  num_cores=NC, num_subcores=NS)
    fn = pl.kernel(
        functools.partial(_gather_body, n_rows_per_worker=n_per_worker),
        out_type=jax.ShapeDtypeStruct((n // SUB, SUB, DIM), dt),
        mesh=mesh,
        compiler_params=pltpu.CompilerParams(needs_layout_passes=False,
                                             use_tc_tiling_on_sc=False),
        scratch_types=[
            pltpu.VMEM((NBUF, SUBS_PER_BLK, SUB), jnp.int32),
            pltpu.VMEM((NBUF, SUBS_PER_BLK, SUB, DIM), dt),
            pltpu.SemaphoreType.DMA,
            pltpu.SemaphoreType.DMA,
            pltpu.SemaphoreType.DMA,
        ],
    )
    return fn(x2, table)


def _norm_kernel(emb_ref, gamma_ref, beta_ref, out_ref):
    x = emb_ref[...].astype(jnp.float32)
    mean = jnp.mean(x, axis=-1, keepdims=True)
    var = jnp.mean(x * x, axis=-1, keepdims=True) - mean * mean
    norm = (x - mean) * lax.rsqrt(var + EPS)
    out_ref[...] = norm * gamma_ref[...] + beta_ref[...]


def _tc_norm(emb, gamma, beta):
    n = emb.shape[0]
    grid = (n // TC_BLK,)
    return pl.pallas_call(
        _norm_kernel,
        grid=grid,
        in_specs=[
            pl.BlockSpec((TC_BLK, DIM), lambda i: (i, 0)),
            pl.BlockSpec((1, DIM), lambda i: (0, 0)),
            pl.BlockSpec((1, DIM), lambda i: (0, 0)),
        ],
        out_specs=pl.BlockSpec((TC_BLK, DIM), lambda i: (i, 0)),
        out_shape=jax.ShapeDtypeStruct((n, DIM), jnp.float32),
        compiler_params=pltpu.CompilerParams(
            dimension_semantics=("parallel",)),
    )(emb, gamma.reshape(1, DIM), beta.reshape(1, DIM))


def kernel(x, table, gamma, beta):
    B, L = x.shape
    n = B * L
    assert n % (NW * R) == 0, (B, L)
    x2 = x.reshape(n // SUB, SUB).astype(jnp.int32)
    raw = _sc_gather(x2, table.astype(jnp.bfloat16), n).reshape(n, DIM)
    out = _tc_norm(raw, gamma, beta)
    return out.reshape(B, L, DIM)


# speedup vs baseline: 1.2180x; 1.2180x over previous
import functools

import jax
import jax.numpy as jnp
from jax import lax
from jax.experimental import pallas as pl
from jax.experimental.pallas import tpu as pltpu
from jax.experimental.pallas import tpu_sc as plsc

DIM = 32
EPS = 1e-5

NC = 2
NS = 16
LANES = 16
NW = NC * NS

SUB = 512
R = SUB
NBUF = 4
WIDE = 128
RPB = R * DIM // WIDE

TC_BLK = 4096


def _gather_body(x_hbm, table_hbm, out_hbm, idx_v, rows_v, sem_g, sem_w,
                 sem_i, n_rows_per_worker):
    wid = lax.axis_index("s") * NC + lax.axis_index("c")
    nb = n_rows_per_worker // R
    sub_base0 = wid * (n_rows_per_worker // SUB)
    wrow_base0 = wid * (n_rows_per_worker * DIM // WIDE)

    def idx_src(b):
        return x_hbm.at[pl.ds(sub_base0 + b, 1)]

    def fire_gather(b):
        q = b & (NBUF - 1)
        pltpu.async_copy(table_hbm.at[idx_v.at[q, 0]], rows_v.at[q, 0], sem_g)

    def wait_gather(b):
        p = b & (NBUF - 1)
        pltpu.make_async_copy(table_hbm.at[idx_v.at[p, 0]],
                              rows_v.at[p, 0], sem_g).wait()

    def wo_refs(b):
        p = b & (NBUF - 1)
        src = rows_v.at[p]
        dst = out_hbm.at[pl.ds(sub_base0 + b, 1)]
        return src, dst

    def wait_writeout():
        src, dst = wo_refs(0)
        pltpu.make_async_copy(src, dst, sem_w).wait()

    pltpu.sync_copy(idx_src(0), idx_v.at[0])
    pltpu.async_copy(idx_src(1), idx_v.at[1], sem_i)
    fire_gather(0)

    def block(b, carry):
        @pl.when(b + 1 < nb)
        def _prefetch():
            @pl.when(b >= NBUF - 1)
            def _():
                wait_writeout()
            pltpu.make_async_copy(idx_src(b + 1),
                                  idx_v.at[(b + 1) & (NBUF - 1)],
                                  sem_i).wait()

            @pl.when(b + 2 < nb)
            def _():
                pltpu.async_copy(idx_src(b + 2),
                                 idx_v.at[(b + 2) & (NBUF - 1)], sem_i)
            fire_gather(b + 1)

        wait_gather(b)
        src, dst = wo_refs(b)
        pltpu.async_copy(src, dst, sem_w)
        return carry

    lax.fori_loop(0, nb, block, None, unroll=False)
    for _ in range(NBUF):
        wait_writeout()


def _sc_gather(x2, table, n):
    n_per_worker = n // NW
    mesh = plsc.VectorSubcoreMesh(core_axis_name="c", subcore_axis_name="s",
                                  num_cores=NC, num_subcores=NS)
    fn = pl.kernel(
        functools.partial(_gather_body, n_rows_per_worker=n_per_worker),
        out_type=jax.ShapeDtypeStruct((n // SUB, SUB, DIM), jnp.float32),
        mesh=mesh,
        compiler_params=pltpu.CompilerParams(needs_layout_passes=False,
                                             use_tc_tiling_on_sc=False),
        scratch_types=[
            pltpu.VMEM((NBUF, 1, SUB), jnp.int32),
            pltpu.VMEM((NBUF, 1, SUB, DIM), jnp.float32),
            pltpu.SemaphoreType.DMA,
            pltpu.SemaphoreType.DMA,
            pltpu.SemaphoreType.DMA,
        ],
    )
    return fn(x2, table)


def _norm_kernel(emb_ref, gamma_ref, beta_ref, out_ref):
    x = emb_ref[...]
    row_seg = lax.broadcasted_iota(jnp.int32, (WIDE, WIDE), 0) // DIM
    col_seg = lax.broadcasted_iota(jnp.int32, (WIDE, WIDE), 1) // DIM
    seg = (row_seg == col_seg).astype(jnp.float32)
    dot = functools.partial(jax.lax.dot_general,
                            dimension_numbers=(((1,), (0,)), ((), ())),
                            precision=lax.Precision.HIGHEST,
                            preferred_element_type=jnp.float32)
    s = dot(x, seg)
    sq = dot(x * x, seg)
    mean = s * (1.0 / DIM)
    var = sq * (1.0 / DIM) - mean * mean
    norm = (x - mean) * lax.rsqrt(var + EPS)
    out_ref[...] = norm * gamma_ref[...] + beta_ref[...]


def _tc_norm(emb, gamma, beta):
    nw = emb.shape[0]
    gt = jnp.tile(gamma.reshape(1, DIM), (1, WIDE // DIM))
    bt = jnp.tile(beta.reshape(1, DIM), (1, WIDE // DIM))
    return pl.pallas_call(
        _norm_kernel,
        grid=(nw // TC_BLK,),
        in_specs=[
            pl.BlockSpec((TC_BLK, WIDE), lambda i: (i, 0)),
            pl.BlockSpec((1, WIDE), lambda i: (0, 0)),
            pl.BlockSpec((1, WIDE), lambda i: (0, 0)),
        ],
        out_specs=pl.BlockSpec((TC_BLK, WIDE), lambda i: (i, 0)),
        out_shape=jax.ShapeDtypeStruct((nw, WIDE), jnp.float32),
        compiler_params=pltpu.CompilerParams(
            dimension_semantics=("parallel",)),
    )(emb, gt, bt)


def kernel(x, table, gamma, beta):
    B, L = x.shape
    n = B * L
    assert n % (NW * R) == 0, (B, L)
    x2 = x.reshape(n // SUB, SUB).astype(jnp.int32)
    raw = _sc_gather(x2, table, n).reshape(n * DIM // WIDE, WIDE)
    out = _tc_norm(raw, gamma, beta)
    return out.reshape(B, L, DIM)


# speedup vs baseline: 1.2205x; 1.0020x over previous
import functools

import jax
import jax.numpy as jnp
from jax import lax
from jax.experimental import pallas as pl
from jax.experimental.pallas import tpu as pltpu
from jax.experimental.pallas import tpu_sc as plsc

DIM = 32
EPS = 1e-5

NC = 2
NS = 16
LANES = 16
NW = NC * NS

SUB = 512
R = SUB
NBUF = 4
WIDE = 128
RPB = R * DIM // WIDE

TC_BLK = 8192


def _gather_body(x_hbm, table_hbm, out_hbm, idx_v, rows_v, sem_g, sem_w,
                 sem_i, n_rows_per_worker):
    wid = lax.axis_index("s") * NC + lax.axis_index("c")
    nb = n_rows_per_worker // R
    sub_base0 = wid * (n_rows_per_worker // SUB)
    wrow_base0 = wid * (n_rows_per_worker * DIM // WIDE)

    def idx_src(b):
        return x_hbm.at[pl.ds(sub_base0 + b, 1)]

    def fire_gather(b):
        q = b & (NBUF - 1)
        pltpu.async_copy(table_hbm.at[idx_v.at[q, 0]], rows_v.at[q, 0], sem_g)

    def wait_gather(b):
        p = b & (NBUF - 1)
        pltpu.make_async_copy(table_hbm.at[idx_v.at[p, 0]],
                              rows_v.at[p, 0], sem_g).wait()

    def wo_refs(b):
        p = b & (NBUF - 1)
        src = rows_v.at[p]
        dst = out_hbm.at[pl.ds(sub_base0 + b, 1)]
        return src, dst

    def wait_writeout():
        src, dst = wo_refs(0)
        pltpu.make_async_copy(src, dst, sem_w).wait()

    pltpu.sync_copy(idx_src(0), idx_v.at[0])
    pltpu.async_copy(idx_src(1), idx_v.at[1], sem_i)
    fire_gather(0)

    def block(b, carry):
        @pl.when(b + 1 < nb)
        def _prefetch():
            @pl.when(b >= NBUF - 1)
            def _():
                wait_writeout()
            pltpu.make_async_copy(idx_src(b + 1),
                                  idx_v.at[(b + 1) & (NBUF - 1)],
                                  sem_i).wait()

            @pl.when(b + 2 < nb)
            def _():
                pltpu.async_copy(idx_src(b + 2),
                                 idx_v.at[(b + 2) & (NBUF - 1)], sem_i)
            fire_gather(b + 1)

        wait_gather(b)
        src, dst = wo_refs(b)
        pltpu.async_copy(src, dst, sem_w)
        return carry

    lax.fori_loop(0, nb, block, None, unroll=False)
    for _ in range(NBUF):
        wait_writeout()


def _sc_gather(x2, table, n):
    n_per_worker = n // NW
    mesh = plsc.VectorSubcoreMesh(core_axis_name="c", subcore_axis_name="s",
                                  num_cores=NC, num_subcores=NS)
    fn = pl.kernel(
        functools.partial(_gather_body, n_rows_per_worker=n_per_worker),
        out_type=jax.ShapeDtypeStruct((n // SUB, SUB, DIM), jnp.float32),
        mesh=mesh,
        compiler_params=pltpu.CompilerParams(needs_layout_passes=False,
                                             use_tc_tiling_on_sc=False),
        scratch_types=[
            pltpu.VMEM((NBUF, 1, SUB), jnp.int32),
            pltpu.VMEM((NBUF, 1, SUB, DIM), jnp.float32),
            pltpu.SemaphoreType.DMA,
            pltpu.SemaphoreType.DMA,
            pltpu.SemaphoreType.DMA,
        ],
    )
    return fn(x2, table)


def _norm_kernel(emb_ref, gamma_ref, beta_ref, out_ref):
    x = emb_ref[...]
    row_seg = lax.broadcasted_iota(jnp.int32, (WIDE, WIDE), 0) // DIM
    col_seg = lax.broadcasted_iota(jnp.int32, (WIDE, WIDE), 1) // DIM
    seg = (row_seg == col_seg).astype(jnp.float32)
    dot = functools.partial(jax.lax.dot_general,
                            dimension_numbers=(((1,), (0,)), ((), ())),
                            precision=lax.Precision.HIGHEST,
                            preferred_element_type=jnp.float32)
    s = dot(x, seg)
    sq = dot(x * x, seg)
    mean = s * (1.0 / DIM)
    var = sq * (1.0 / DIM) - mean * mean
    norm = (x - mean) * lax.rsqrt(var + EPS)
    out_ref[...] = norm * gamma_ref[...] + beta_ref[...]


def _tc_norm(emb, gamma, beta):
    nw = emb.shape[0]
    gt = jnp.tile(gamma.reshape(1, DIM), (1, WIDE // DIM))
    bt = jnp.tile(beta.reshape(1, DIM), (1, WIDE // DIM))
    return pl.pallas_call(
        _norm_kernel,
        grid=(nw // TC_BLK,),
        in_specs=[
            pl.BlockSpec((TC_BLK, WIDE), lambda i: (i, 0)),
            pl.BlockSpec((1, WIDE), lambda i: (0, 0)),
            pl.BlockSpec((1, WIDE), lambda i: (0, 0)),
        ],
        out_specs=pl.BlockSpec((TC_BLK, WIDE), lambda i: (i, 0)),
        out_shape=jax.ShapeDtypeStruct((nw, WIDE), jnp.float32),
        compiler_params=pltpu.CompilerParams(
            dimension_semantics=("parallel",)),
    )(emb, gt, bt)


def kernel(x, table, gamma, beta):
    B, L = x.shape
    n = B * L
    assert n % (NW * R) == 0, (B, L)
    x2 = x.reshape(n // SUB, SUB).astype(jnp.int32)
    raw = _sc_gather(x2, table, n).reshape(n * DIM // WIDE, WIDE)
    out = _tc_norm(raw, gamma, beta)
    return out.reshape(B, L, DIM)
